# Initial kernel scaffold; baseline (speedup 1.0000x reference)
import jax
import jax.numpy as jnp
from jax.experimental import pallas as pl


def kernel(x, edge_index0, values0, edge_index1, values1, kernel, bias):
    raise NotImplementedError("write your pallas kernel here")


# speedup vs baseline: 1.3112x; 1.3112x over previous
import functools

import jax
import jax.numpy as jnp
from jax import lax
from jax.experimental import pallas as pl
from jax.experimental.pallas import tpu as pltpu
from jax.experimental.pallas import tpu_sc as plsc

B = 8
N = 10000
D = 128
U = 128
E = 320000
M = B * N

NUM_TECS = 16
EPT = E // NUM_TECS
CHUNK = 80
NCHUNK = EPT // CHUNK
RPT = 624
TAIL = N - RPT * NUM_TECS
ZROWS = 128

_BM = 2000


def _mm2_body(x_ref, k1_ref, k2_ref, z1_ref, z2_ref):
    xb = x_ref[...]
    z1_ref[...] = jnp.dot(xb, k1_ref[...], preferred_element_type=jnp.float32)
    z2_ref[...] = jnp.dot(xb, k2_ref[...], preferred_element_type=jnp.float32)


def _mmadd_body(x_ref, s_ref, k0_ref, b_ref, o_ref):
    o_ref[...] = (jnp.dot(x_ref[...], k0_ref[...],
                          preferred_element_type=jnp.float32)
                  + s_ref[...] + b_ref[...][0:1, :])


def _sc_body(z1_hbm, z2_hbm, r0_hbm, c0_hbm, v0_hbm, r1_hbm, c1_hbm, v1_hbm,
             out_hbm, acc, colb, rowb, valb, gbuf, zbuf, sem):
    cid = lax.axis_index("c")
    sid = lax.axis_index("s")
    base = sid * RPT

    def zloop(i, _):
        for j in range(U // 16):
            zbuf[i, pl.ds(j * 16, 16)] = jnp.zeros((16,), jnp.float32)
        return 0
    lax.fori_loop(0, ZROWS, zloop, 0)

    for bi in range(B // 2):
        b = cid * (B // 2) + bi
        bN = b * N

        for k in range(RPT // ZROWS):
            pltpu.sync_copy(zbuf, acc.at[pl.ds(base + k * ZROWS, ZROWS)])
        rem = RPT % ZROWS
        if rem:
            pltpu.sync_copy(zbuf.at[pl.ds(0, rem)],
                            acc.at[pl.ds(base + RPT - rem, rem)])

        @pl.when(sid == NUM_TECS - 1)
        def _zero_tail():
            pltpu.sync_copy(zbuf.at[pl.ds(0, TAIL)],
                            acc.at[pl.ds(RPT * NUM_TECS, TAIL)])
        plsc.subcore_barrier()

        for z_hbm, r_hbm, c_hbm, v_hbm in (
                (z1_hbm, r0_hbm, c0_hbm, v0_hbm),
                (z2_hbm, r1_hbm, c1_hbm, v1_hbm)):
            ebase = sid * EPT

            def chunk_body(k, _, z_hbm=z_hbm, r_hbm=r_hbm, c_hbm=c_hbm,
                           v_hbm=v_hbm, bN=bN, ebase=ebase):
                off = ebase + k * CHUNK
                pltpu.sync_copy(c_hbm.at[pl.ds(off, CHUNK)], colb)
                pltpu.sync_copy(r_hbm.at[pl.ds(off, CHUNK)], rowb)
                pltpu.sync_copy(v_hbm.at[pl.ds(off, CHUNK)], valb)
                for j in range(CHUNK // 16):
                    colb[pl.ds(j * 16, 16)] = colb[pl.ds(j * 16, 16)] + bN
                pltpu.async_copy(z_hbm.at[colb], gbuf, sem).wait()

                def srow(t, _):
                    valv = valb[pl.ds(t * 16, 16)]
                    for i in range(16):
                        r = t * 16 + i
                        v = valv[i]
                        for j in range(U // 16):
                            gbuf[r, pl.ds(j * 16, 16)] = (
                                gbuf[r, pl.ds(j * 16, 16)] * v)
                    return 0
                lax.fori_loop(0, CHUNK // 16, srow, 0)
                pltpu.sync_copy(gbuf, acc.at[rowb], add=True)
                return 0

            lax.fori_loop(0, NCHUNK, chunk_body, 0)
        plsc.subcore_barrier()
        pltpu.sync_copy(acc.at[pl.ds(base, RPT)],
                        out_hbm.at[pl.ds(bN + base, RPT)])

        @pl.when(sid == NUM_TECS - 1)
        def _flush_tail():
            pltpu.sync_copy(acc.at[pl.ds(RPT * NUM_TECS, TAIL)],
                            out_hbm.at[pl.ds(bN + RPT * NUM_TECS, TAIL)])


_sc_spmm = functools.partial(
    pl.kernel,
    out_type=jax.ShapeDtypeStruct((M, U), jnp.float32),
    mesh=plsc.VectorSubcoreMesh(core_axis_name="c", subcore_axis_name="s"),
    scratch_types=[
        pltpu.VMEM_SHARED((N, U), jnp.float32),
        pltpu.VMEM((CHUNK,), jnp.int32),
        pltpu.VMEM((CHUNK,), jnp.int32),
        pltpu.VMEM((CHUNK,), jnp.float32),
        pltpu.VMEM((CHUNK, U), jnp.float32),
        pltpu.VMEM((ZROWS, U), jnp.float32),
        pltpu.SemaphoreType.DMA,
    ],
)(_sc_body)


def kernel(x, edge_index0, values0, edge_index1, values1, kernel, bias):
    xf = x.reshape(M, D)
    kw = kernel.reshape(D, 3, U)
    k0, k1, k2 = kw[:, 0, :], kw[:, 1, :], kw[:, 2, :]

    z1, z2 = pl.pallas_call(
        _mm2_body,
        grid=(M // _BM,),
        in_specs=[
            pl.BlockSpec((_BM, D), lambda i: (i, 0)),
            pl.BlockSpec((D, U), lambda i: (0, 0)),
            pl.BlockSpec((D, U), lambda i: (0, 0)),
        ],
        out_specs=[
            pl.BlockSpec((_BM, U), lambda i: (i, 0)),
            pl.BlockSpec((_BM, U), lambda i: (i, 0)),
        ],
        out_shape=[
            jax.ShapeDtypeStruct((M, U), jnp.float32),
            jax.ShapeDtypeStruct((M, U), jnp.float32),
        ],
    )(xf, k1, k2)

    s = _sc_spmm(z1, z2,
                 edge_index0[0], edge_index0[1], values0,
                 edge_index1[0], edge_index1[1], values1)

    bias2 = jnp.broadcast_to(bias, (8, U))
    out = pl.pallas_call(
        _mmadd_body,
        grid=(M // _BM,),
        in_specs=[
            pl.BlockSpec((_BM, D), lambda i: (i, 0)),
            pl.BlockSpec((_BM, U), lambda i: (i, 0)),
            pl.BlockSpec((D, U), lambda i: (0, 0)),
            pl.BlockSpec((8, U), lambda i: (0, 0)),
        ],
        out_specs=pl.BlockSpec((_BM, U), lambda i: (i, 0)),
        out_shape=jax.ShapeDtypeStruct((M, U), jnp.float32),
    )(xf, s, k0, bias2)

    return out.reshape(B, N, U)


# speedup vs baseline: 2.0407x; 1.5564x over previous
import functools

import jax
import jax.numpy as jnp
from jax import lax
from jax.experimental import pallas as pl
from jax.experimental.pallas import tpu as pltpu
from jax.experimental.pallas import tpu_sc as plsc

B = 8
N = 10000
D = 128
U = 128
E = 320000
M = B * N

NUM_TECS = 16
CHUNK = 128
NCHUNK = 158
EPT = NCHUNK * CHUNK
EPAD = NUM_TECS * EPT
RPT = 624
TAIL = N - RPT * NUM_TECS

_BM = 2000


def _mm2_body(x_ref, k1_ref, k2_ref, z1_ref, z2_ref):
    xb = x_ref[...]
    z1_ref[...] = jnp.dot(xb, k1_ref[...], preferred_element_type=jnp.float32)
    z2_ref[...] = jnp.dot(xb, k2_ref[...], preferred_element_type=jnp.float32)


def _mmadd_body(x_ref, s_ref, k0_ref, b_ref, o_ref):
    o_ref[...] = (jnp.dot(x_ref[...], k0_ref[...],
                          preferred_element_type=jnp.float32)
                  + s_ref[...] + b_ref[...][0:1, :])


def _sc_body(z1_hbm, z2_hbm, p0_hbm, v0_hbm, p1_hbm, v1_hbm, out_hbm,
             acc, ring_a, ring_b, vring_a, vring_b, cola, colb,
             gbuf_a, gbuf_b, semg_a, semg_b):
    cid = lax.axis_index("c")
    sid = lax.axis_index("s")
    base = sid * RPT
    rings = (ring_a, ring_b)
    vrings = (vring_a, vring_b)
    colbufs = (cola, colb)
    gbufs = (gbuf_a, gbuf_b)
    sems = (semg_a, semg_b)

    def batch_body(bi, _):
        b = cid * (B // 2) + bi
        bN = b * N

        def zloop(i, _):
            for j in range(U // 16):
                gbuf_a[i, pl.ds(j * 16, 16)] = jnp.zeros((16,), jnp.float32)
            return 0
        lax.fori_loop(0, CHUNK, zloop, 0)
        for k in range(RPT // CHUNK):
            pltpu.sync_copy(gbuf_a, acc.at[pl.ds(base + k * CHUNK, CHUNK)])
        rem = RPT % CHUNK
        if rem:
            pltpu.sync_copy(gbuf_a.at[pl.ds(0, rem)],
                            acc.at[pl.ds(base + RPT - rem, rem)])

        @pl.when(sid == NUM_TECS - 1)
        def _zero_tail():
            pltpu.sync_copy(gbuf_a.at[pl.ds(0, TAIL)],
                            acc.at[pl.ds(RPT * NUM_TECS, TAIL)])
        plsc.subcore_barrier()

        for z_hbm, p_hbm, v_hbm in ((z1_hbm, p0_hbm, v0_hbm),
                                    (z2_hbm, p1_hbm, v1_hbm)):

            def stage_and_gather(k, p, z_hbm=z_hbm, p_hbm=p_hbm,
                                 v_hbm=v_hbm, bN=bN):
                rg = rings[p]
                cb = colbufs[p]
                pltpu.sync_copy(p_hbm.at[sid, k], rg)
                pltpu.sync_copy(v_hbm.at[sid, k], vrings[p])
                for j in range(CHUNK // 16):
                    cb[pl.ds(j * 16, 16)] = rg[0, pl.ds(j * 16, 16)] + bN
                pltpu.async_copy(z_hbm.at[cb], gbufs[p], sems[p])

            def process(k, p, z_hbm=z_hbm):
                pltpu.make_async_copy(
                    z_hbm.at[colbufs[p]], gbufs[p], sems[p]).wait()
                rg = rings[p]
                gb = gbufs[p]

                def srow(t, _):
                    valv = vrings[p][pl.ds(t * 16, 16)]
                    for i in range(16):
                        r = t * 16 + i
                        v = valv[i]
                        for j in range(U // 16):
                            gb[r, pl.ds(j * 16, 16)] = (
                                gb[r, pl.ds(j * 16, 16)] * v)
                    return 0
                lax.fori_loop(0, CHUNK // 16, srow, 0)
                pltpu.sync_copy(gb, acc.at[rg.at[1]], add=True)

            stage_and_gather(0, 0)

            def pair_body(k2, _):
                k = k2 * 2
                stage_and_gather(k + 1, 1)
                process(k, 0)
                stage_and_gather(k + 2, 0)
                process(k + 1, 1)
                return 0
            lax.fori_loop(0, NCHUNK // 2 - 1, pair_body, 0)
            stage_and_gather(NCHUNK - 1, 1)
            process(NCHUNK - 2, 0)
            process(NCHUNK - 1, 1)
        plsc.subcore_barrier()
        pltpu.sync_copy(acc.at[pl.ds(base, RPT)],
                        out_hbm.at[pl.ds(bN + base, RPT)])

        @pl.when(sid == NUM_TECS - 1)
        def _flush_tail():
            pltpu.sync_copy(acc.at[pl.ds(RPT * NUM_TECS, TAIL)],
                            out_hbm.at[pl.ds(bN + RPT * NUM_TECS, TAIL)])
        return 0

    lax.fori_loop(0, B // 2, batch_body, 0)


_sc_spmm = functools.partial(
    pl.kernel,
    out_type=jax.ShapeDtypeStruct((M, U), jnp.float32),
    mesh=plsc.VectorSubcoreMesh(core_axis_name="c", subcore_axis_name="s"),
    scratch_types=[
        pltpu.VMEM_SHARED((N, U), jnp.float32),
        pltpu.VMEM((2, CHUNK), jnp.int32),
        pltpu.VMEM((2, CHUNK), jnp.int32),
        pltpu.VMEM((CHUNK,), jnp.float32),
        pltpu.VMEM((CHUNK,), jnp.float32),
        pltpu.VMEM((CHUNK,), jnp.int32),
        pltpu.VMEM((CHUNK,), jnp.int32),
        pltpu.VMEM((CHUNK, U), jnp.float32),
        pltpu.VMEM((CHUNK, U), jnp.float32),
        pltpu.SemaphoreType.DMA,
        pltpu.SemaphoreType.DMA,
    ],
)(_sc_body)


def _pack_edges(edge_index, values):
    pad = EPAD - E
    cols = jnp.pad(edge_index[1], (0, pad))
    rows = jnp.pad(edge_index[0], (0, pad))
    vals = jnp.pad(values, (0, pad))
    packed = jnp.stack([cols, rows], 0).reshape(2, NUM_TECS, NCHUNK, CHUNK)
    return (jnp.transpose(packed, (1, 2, 0, 3)),
            vals.reshape(NUM_TECS, NCHUNK, CHUNK))


def kernel(x, edge_index0, values0, edge_index1, values1, kernel, bias):
    xf = x.reshape(M, D)
    kw = kernel.reshape(D, 3, U)
    k0, k1, k2 = kw[:, 0, :], kw[:, 1, :], kw[:, 2, :]

    z1, z2 = pl.pallas_call(
        _mm2_body,
        grid=(M // _BM,),
        in_specs=[
            pl.BlockSpec((_BM, D), lambda i: (i, 0)),
            pl.BlockSpec((D, U), lambda i: (0, 0)),
            pl.BlockSpec((D, U), lambda i: (0, 0)),
        ],
        out_specs=[
            pl.BlockSpec((_BM, U), lambda i: (i, 0)),
            pl.BlockSpec((_BM, U), lambda i: (i, 0)),
        ],
        out_shape=[
            jax.ShapeDtypeStruct((M, U), jnp.float32),
            jax.ShapeDtypeStruct((M, U), jnp.float32),
        ],
    )(xf, k1, k2)

    p0, v0 = _pack_edges(edge_index0, values0)
    p1, v1 = _pack_edges(edge_index1, values1)
    s = _sc_spmm(z1, z2, p0, v0, p1, v1)

    bias2 = jnp.broadcast_to(bias, (8, U))
    out = pl.pallas_call(
        _mmadd_body,
        grid=(M // _BM,),
        in_specs=[
            pl.BlockSpec((_BM, D), lambda i: (i, 0)),
            pl.BlockSpec((_BM, U), lambda i: (i, 0)),
            pl.BlockSpec((D, U), lambda i: (0, 0)),
            pl.BlockSpec((8, U), lambda i: (0, 0)),
        ],
        out_specs=pl.BlockSpec((_BM, U), lambda i: (i, 0)),
        out_shape=jax.ShapeDtypeStruct((M, U), jnp.float32),
    )(xf, s, k0, bias2)

    return out.reshape(B, N, U)
